# baseline (speedup 1.0000x reference)
import functools

import jax
import jax.numpy as jnp
from jax import lax
from jax.experimental import pallas as pl
from jax.experimental.pallas import tpu as pltpu
from jax.experimental.pallas import tpu_sc as plsc

NCLS = 19
HW = 512 * 512
NBINS = 368

ROWS = 256


def _argmax_body(p_ref, t_ref, o_ref):
    m = p_ref[0, 0]
    pred = jnp.zeros((ROWS, 512), jnp.int32)
    for c in range(1, NCLS):
        xc = p_ref[0, c]
        gt = xc > m
        m = jnp.where(gt, xc, m)
        pred = jnp.where(gt, c, pred)
    lbl = t_ref[0]
    valid = (lbl >= 0) & (lbl < NCLS)
    idx = jnp.where(valid, lbl * NCLS + pred, NCLS * NCLS)
    o_ref[...] = idx.reshape(ROWS * 4, 128)


def _argmax_idx(preds, target, b0, nb):
    _, _, h, w = preds.shape
    nj = h // ROWS
    return pl.pallas_call(
        _argmax_body,
        grid=(nb, nj),
        in_specs=[
            pl.BlockSpec((1, NCLS, ROWS, w), lambda i, j: (b0 + i, 0, j, 0)),
            pl.BlockSpec((1, ROWS, w), lambda i, j: (b0 + i, j, 0)),
        ],
        out_specs=pl.BlockSpec((ROWS * 4, 128), lambda i, j: (i * nj + j, 0)),
        out_shape=jax.ShapeDtypeStruct((nb * h * w // 128, 128), jnp.int32),
        compiler_params=pltpu.CompilerParams(
            dimension_semantics=("parallel", "parallel")),
    )(preds, target)


def _make_sc_body(nimg):
    epw = nimg * HW // 32
    wpi = 32 // nimg
    ipc = nimg // 2

    nch = 4
    crows = epw // 128 // nch

    def body(idx_hbm, out_hbm, idx_a, idx_b, hist_v, red_v, part_v, out_v,
             shared, sem_a, sem_b):
        cid = lax.axis_index("c")
        sid = lax.axis_index("s")
        wid = cid * 16 + sid
        base = wid * epw // 128
        bufs = [idx_a, idx_b]
        sems = [sem_a, sem_b]

        h = pltpu.async_copy(
            idx_hbm.at[pl.ds(pl.multiple_of(base, 8), crows)], idx_a, sem_a)

        zeros16 = jnp.zeros((16,), jnp.float32)

        def zbody(i, carry):
            hist_v[pl.ds(i * 16, 16)] = zeros16
            return carry

        lax.fori_loop(0, NBINS, zbody, 0, unroll=8)

        lane_base = lax.iota(jnp.int32, 16) * NBINS
        ones16 = jnp.ones((16,), jnp.float32)

        def scatter_chunk(buf):
            def sbody(i, carry):
                vs = [buf[i * 4 + rr, pl.ds(k * 16, 16)]
                      for rr in range(4) for k in range(8)]
                ts = [lane_base + v for v in vs]
                for t in ts:
                    plsc.addupdate_scatter(hist_v, [t], ones16)
                return carry

            lax.fori_loop(0, crows // 4, sbody, 0)

        for ch in range(nch):
            nxt = None
            if ch + 1 < nch:
                nxt = pltpu.async_copy(
                    idx_hbm.at[pl.ds(
                        pl.multiple_of(base + (ch + 1) * crows, 8), crows)],
                    bufs[(ch + 1) % 2], sems[(ch + 1) % 2])
            h.wait()
            scatter_chunk(bufs[ch % 2])
            h = nxt

        def rbody(j, carry):
            acc = hist_v[pl.ds(j * 16, 16)]
            for k in range(1, 16):
                acc = acc + hist_v[pl.ds(k * NBINS + j * 16, 16)]
            red_v[pl.ds(j * 16, 16)] = acc
            return carry

        lax.fori_loop(0, NBINS // 16, rbody, 0)

        pltpu.sync_copy(red_v, shared.at[pl.ds(sid * NBINS, NBINS)])
        plsc.subcore_barrier()

        @pl.when(sid % wpi == 0)
        def _leader():
            img = cid * ipc + sid // wpi
            pltpu.sync_copy(shared.at[pl.ds(sid * NBINS, wpi * NBINS)],
                            part_v)

            def hbody(j, carry):
                acc = part_v[pl.ds(j * 16, 16)]
                for k in range(1, wpi):
                    acc = acc + part_v[pl.ds(k * NBINS + j * 16, 16)]
                red_v[pl.ds(j * 16, 16)] = acc
                return carry

            lax.fori_loop(0, NBINS // 16, hbody, 0)

            lanes = lax.iota(jnp.int32, 16)
            for c2 in range(2):
                cls = jnp.minimum(lanes + 16 * c2, NCLS - 1)
                row = plsc.load_gather(red_v, [cls * NCLS])
                col = plsc.load_gather(red_v, [cls])
                for j in range(1, NCLS):
                    row = row + plsc.load_gather(red_v, [cls * NCLS + j])
                    col = col + plsc.load_gather(red_v, [j * NCLS + cls])
                diag = plsc.load_gather(red_v, [cls * (NCLS + 1)])
                union = jnp.maximum(row + col - diag, 1.0)
                out_v[pl.ds(16 * c2, 16)] = diag / union
            pltpu.sync_copy(out_v, out_hbm.at[pl.ds(img * 32, 32)])

    return body


_sc_hist_cache = {}


def _sc_hist(nimg):
    if nimg not in _sc_hist_cache:
        epw = nimg * HW // 32
        wpi = 32 // nimg
        _sc_hist_cache[nimg] = functools.partial(
            pl.kernel,
            mesh=plsc.VectorSubcoreMesh(core_axis_name="c",
                                        subcore_axis_name="s"),
            out_type=jax.ShapeDtypeStruct((nimg * 32,), jnp.float32),
            scratch_types=[
                pltpu.VMEM((epw // 128 // 4, 128), jnp.int32),
                pltpu.VMEM((epw // 128 // 4, 128), jnp.int32),
                pltpu.VMEM((16 * NBINS,), jnp.float32),
                pltpu.VMEM((NBINS,), jnp.float32),
                pltpu.VMEM((wpi * NBINS,), jnp.float32),
                pltpu.VMEM((32,), jnp.float32),
                pltpu.VMEM_SHARED((16 * NBINS,), jnp.float32),
                pltpu.SemaphoreType.DMA,
                pltpu.SemaphoreType.DMA,
            ],
            compiler_params=pltpu.CompilerParams(needs_layout_passes=False),
        )(_make_sc_body(nimg))
    return _sc_hist_cache[nimg]


def kernel(preds, target):
    b = preds.shape[0]
    idx = _argmax_idx(preds, target, 0, b)
    out = _sc_hist(b)(idx)
    return out.reshape(b, 32)[:, :NCLS]
